# Initial kernel scaffold; baseline (speedup 1.0000x reference)
import jax
import jax.numpy as jnp
from jax.experimental import pallas as pl


def kernel(x, edge_index, W1_0, b1_0, W2_0, b2_0, W1s, b1s, W2s, b2s, Wm, bm, Ws, bs):
    raise NotImplementedError("write your pallas kernel here")


# speedup vs baseline: 25.0074x; 25.0074x over previous
import functools
import jax
import jax.numpy as jnp
from jax import lax
from jax.experimental import pallas as pl
from jax.experimental.pallas import tpu as pltpu
from jax.experimental.pallas import tpu_sc as plsc

N = 10000
E = 320000
HID = 16

NC = 2
NS = 16
NW = NC * NS
EPW = E // NW
CH = 2000
NCHUNK = EPW // CH
NP = 10240
RPT = NP // NS


def _agg_body(h_hbm, src_hbm, dst_hbm, out_hbm, sidx_v, didx_v, msgs_v, agg_sh, gsem):
    c = lax.axis_index("c")
    s = lax.axis_index("s")
    wid = c * NS + s

    zrow = jnp.zeros((HID,), jnp.float32)

    def zero_local(r, _):
        msgs_v[r, :] = zrow
        return 0

    lax.fori_loop(0, RPT, zero_local, 0)
    pltpu.sync_copy(msgs_v.at[pl.ds(0, RPT)], agg_sh.at[pl.ds(s * RPT, RPT)])
    plsc.subcore_barrier()

    base = wid * EPW

    def chunk(i, _):
        off = base + i * CH
        pltpu.sync_copy(src_hbm.at[pl.ds(off, CH)], sidx_v)
        pltpu.sync_copy(dst_hbm.at[pl.ds(off, CH)], didx_v)
        pltpu.async_copy(h_hbm.at[sidx_v], msgs_v, gsem).wait()
        pltpu.sync_copy(msgs_v, agg_sh.at[didx_v], add=True)
        return 0

    lax.fori_loop(0, NCHUNK, chunk, 0)
    plsc.subcore_barrier()

    full = N // RPT

    @pl.when(s < full)
    def _():
        pltpu.sync_copy(agg_sh.at[pl.ds(s * RPT, RPT)],
                        out_hbm.at[c, pl.ds(s * RPT, RPT)])

    rem = N - full * RPT

    @pl.when(s == full)
    def _():
        pltpu.sync_copy(agg_sh.at[pl.ds(full * RPT, rem)],
                        out_hbm.at[c, pl.ds(full * RPT, rem)])


@jax.jit
def _sc_aggregate(h, src, dst):
    mesh = plsc.VectorSubcoreMesh(core_axis_name="c", subcore_axis_name="s")
    return pl.kernel(
        _agg_body,
        out_type=jax.ShapeDtypeStruct((NC, N, HID), jnp.float32),
        mesh=mesh,
        compiler_params=pltpu.CompilerParams(use_tc_tiling_on_sc=False),
        scratch_types=[
            pltpu.VMEM((CH,), jnp.int32),
            pltpu.VMEM((CH,), jnp.int32),
            pltpu.VMEM((CH, HID), jnp.float32),
            pltpu.VMEM_SHARED((NP, HID), jnp.float32),
            pltpu.SemaphoreType.DMA,
        ],
    )(h, src, dst)


BR = 2000


def _pre_body(x_ref, w_ref, o_ref):
    o_ref[...] = jnp.dot(x_ref[...], w_ref[...], preferred_element_type=jnp.float32)


@jax.jit
def _tc_project(x, w):
    d = x.shape[1]
    return pl.pallas_call(
        _pre_body,
        grid=(N // BR,),
        in_specs=[
            pl.BlockSpec((BR, d), lambda i: (i, 0)),
            pl.BlockSpec((d, HID), lambda i: (0, 0)),
        ],
        out_specs=pl.BlockSpec((BR, HID), lambda i: (i, 0)),
        out_shape=jax.ShapeDtypeStruct((N, HID), jnp.float32),
    )(x, w)


def _mlp0_body(u_ref, agg_ref, b1_ref, w2_ref, b2_ref, o_ref):
    t = u_ref[...] + agg_ref[0] + agg_ref[1] + b1_ref[...]
    a = jnp.maximum(t, 0.0)
    h = jnp.dot(a, w2_ref[...], preferred_element_type=jnp.float32) + b2_ref[...]
    o_ref[...] = jnp.maximum(h, 0.0)


@jax.jit
def _tc_mlp0(u, agg, b1, w2, b2):
    return pl.pallas_call(
        _mlp0_body,
        grid=(N // BR,),
        in_specs=[
            pl.BlockSpec((BR, HID), lambda i: (i, 0)),
            pl.BlockSpec((NC, BR, HID), lambda i: (0, i, 0)),
            pl.BlockSpec((1, HID), lambda i: (0, 0)),
            pl.BlockSpec((HID, HID), lambda i: (0, 0)),
            pl.BlockSpec((1, HID), lambda i: (0, 0)),
        ],
        out_specs=pl.BlockSpec((BR, HID), lambda i: (i, 0)),
        out_shape=jax.ShapeDtypeStruct((N, HID), jnp.float32),
    )(u, agg, b1, w2, b2)


def _mlp_body(h_ref, agg_ref, w1_ref, b1_ref, w2_ref, b2_ref, o_ref, *, relu):
    z = h_ref[...] + agg_ref[0] + agg_ref[1]
    t = jnp.maximum(jnp.dot(z, w1_ref[...], preferred_element_type=jnp.float32)
                    + b1_ref[...], 0.0)
    h = jnp.dot(t, w2_ref[...], preferred_element_type=jnp.float32) + b2_ref[...]
    if relu:
        h = jnp.maximum(h, 0.0)
    o_ref[...] = h


@functools.partial(jax.jit, static_argnames=("relu",))
def _tc_mlp(h, agg, w1, b1, w2, b2, relu):
    return pl.pallas_call(
        functools.partial(_mlp_body, relu=relu),
        grid=(N // BR,),
        in_specs=[
            pl.BlockSpec((BR, HID), lambda i: (i, 0)),
            pl.BlockSpec((NC, BR, HID), lambda i: (0, i, 0)),
            pl.BlockSpec((HID, HID), lambda i: (0, 0)),
            pl.BlockSpec((1, HID), lambda i: (0, 0)),
            pl.BlockSpec((HID, HID), lambda i: (0, 0)),
            pl.BlockSpec((1, HID), lambda i: (0, 0)),
        ],
        out_specs=pl.BlockSpec((BR, HID), lambda i: (i, 0)),
        out_shape=jax.ShapeDtypeStruct((N, HID), jnp.float32),
    )(h, agg, w1, b1, w2, b2)


def _mlp_head_body(h_ref, agg_ref, w1_ref, b1_ref, w2_ref, b2_ref,
                   wm_ref, bm_ref, ws_ref, bs_ref, mean_ref, std_ref):
    z = h_ref[...] + agg_ref[0] + agg_ref[1]
    t = jnp.maximum(jnp.dot(z, w1_ref[...], preferred_element_type=jnp.float32)
                    + b1_ref[...], 0.0)
    h = jnp.dot(t, w2_ref[...], preferred_element_type=jnp.float32) + b2_ref[...]
    mean_ref[...] = jnp.dot(h, wm_ref[...], preferred_element_type=jnp.float32) + bm_ref[...]
    y = jnp.dot(h, ws_ref[...], preferred_element_type=jnp.float32) + bs_ref[...]
    std_ref[...] = jnp.maximum(y, 0.0) + jnp.log1p(jnp.exp(-jnp.abs(y)))


@jax.jit
def _tc_mlp_head(h, agg, w1, b1, w2, b2, wm, bm, ws, bs):
    lat = wm.shape[1]
    return pl.pallas_call(
        _mlp_head_body,
        grid=(N // BR,),
        in_specs=[
            pl.BlockSpec((BR, HID), lambda i: (i, 0)),
            pl.BlockSpec((NC, BR, HID), lambda i: (0, i, 0)),
            pl.BlockSpec((HID, HID), lambda i: (0, 0)),
            pl.BlockSpec((1, HID), lambda i: (0, 0)),
            pl.BlockSpec((HID, HID), lambda i: (0, 0)),
            pl.BlockSpec((1, HID), lambda i: (0, 0)),
            pl.BlockSpec((HID, lat), lambda i: (0, 0)),
            pl.BlockSpec((1, lat), lambda i: (0, 0)),
            pl.BlockSpec((HID, lat), lambda i: (0, 0)),
            pl.BlockSpec((1, lat), lambda i: (0, 0)),
        ],
        out_specs=[
            pl.BlockSpec((BR, lat), lambda i: (i, 0)),
            pl.BlockSpec((BR, lat), lambda i: (i, 0)),
        ],
        out_shape=[
            jax.ShapeDtypeStruct((N, lat), jnp.float32),
            jax.ShapeDtypeStruct((N, lat), jnp.float32),
        ],
    )(h, agg, w1, b1, w2, b2, wm, bm, ws, bs)


def kernel(x, edge_index, W1_0, b1_0, W2_0, b2_0, W1s, b1s, W2s, b2s, Wm, bm, Ws, bs):
    src = jnp.asarray(edge_index[0], jnp.int32)
    dst = jnp.asarray(edge_index[1], jnp.int32)

    u = _tc_project(x, W1_0)
    agg = _sc_aggregate(u, src, dst)
    h = _tc_mlp0(u, agg, b1_0.reshape(1, HID), W2_0, b2_0.reshape(1, HID))

    n_rest = W1s.shape[0]
    for i in range(n_rest - 1):
        agg = _sc_aggregate(h, src, dst)
        h = _tc_mlp(h, agg, W1s[i], b1s[i].reshape(1, HID), W2s[i],
                    b2s[i].reshape(1, HID), relu=True)

    i = n_rest - 1
    agg = _sc_aggregate(h, src, dst)
    lat = Wm.shape[1]
    mean, std = _tc_mlp_head(h, agg, W1s[i], b1s[i].reshape(1, HID), W2s[i],
                             b2s[i].reshape(1, HID), Wm, bm.reshape(1, lat),
                             Ws, bs.reshape(1, lat))
    return (mean, std)
